# baseline (speedup 1.0000x reference)
import functools

import jax
import jax.numpy as jnp
from jax import lax
from jax.experimental import pallas as pl
from jax.experimental.pallas import tpu as pltpu

LANE = 128


def _round_up(x, m):
    return (x + m - 1) // m * m


def _conv_kernel(x_ref, w_ref, b_ref, y_ref, s_ref, *, W, H, NL):
    HP = H + 2
    ROWS = W * HP
    x = x_ref[0]
    y = jnp.zeros((ROWS, w_ref.shape[-1]), jnp.float32) + b_ref[...]
    for kh in range(3):
        for kw in range(3):
            s = kh * HP + kw
            y = y + jnp.dot(x[s:s + ROWS, :], w_ref[kh * 3 + kw],
                            preferred_element_type=jnp.float32)
    col = lax.broadcasted_iota(jnp.int32, (ROWS, 1), 0) % HP
    m = (col < H).astype(jnp.float32)
    y8 = y[:, :NL]
    ym = y8 * m
    s_ref[0, 0:1, :] = jnp.sum(ym, axis=0, keepdims=True)
    s_ref[0, 1:2, :] = jnp.sum(ym * y8, axis=0, keepdims=True)
    y_ref[0] = y8


def _conv_head(x, wmat, bias, *, W, H, NL):
    B, XR, C = x.shape
    CP = wmat.shape[-1]
    ROWS = W * (H + 2)
    b_p = jnp.pad(bias, (0, CP - bias.shape[0])).reshape(1, CP)
    kern = functools.partial(_conv_kernel, W=W, H=H, NL=NL)
    return pl.pallas_call(
        kern,
        out_shape=(jax.ShapeDtypeStruct((B, ROWS, NL), jnp.float32),
                   jax.ShapeDtypeStruct((B, 8, NL), jnp.float32)),
        grid_spec=pltpu.PrefetchScalarGridSpec(
            num_scalar_prefetch=0,
            grid=(B,),
            in_specs=[
                pl.BlockSpec((1, XR, C), lambda b: (b, 0, 0)),
                pl.BlockSpec((9, C, CP), lambda b: (0, 0, 0)),
                pl.BlockSpec((1, CP), lambda b: (0, 0)),
            ],
            out_specs=[pl.BlockSpec((1, ROWS, NL), lambda b: (b, 0, 0)),
                       pl.BlockSpec((1, 8, NL), lambda b: (b, 0, 0))],
        ),
        compiler_params=pltpu.CompilerParams(
            dimension_semantics=("parallel",),
            vmem_limit_bytes=64 * 1024 * 1024),
    )(x, wmat, b_p)


def _gtab_kernel(f0_ref, w_ref, o_ref, *, PT, P):
    o_ref[...] = jnp.zeros_like(o_ref)
    f0 = f0_ref[...]
    for k in range(3):
        o_ref[k, PT:PT + P, :] = jnp.dot(f0, w_ref[k],
                                         preferred_element_type=jnp.float32)


def _gtab(f0, w2, *, PT):
    P, C = f0.shape
    K, _, O = w2.shape
    GR = P + 2 * PT
    kern = functools.partial(_gtab_kernel, PT=PT, P=P)
    return pl.pallas_call(
        kern,
        out_shape=jax.ShapeDtypeStruct((K, GR, O), jnp.float32),
        grid_spec=pltpu.PrefetchScalarGridSpec(
            num_scalar_prefetch=0,
            grid=(1,),
            in_specs=[
                pl.BlockSpec((P, C), lambda i: (0, 0)),
                pl.BlockSpec((K, C, O), lambda i: (0, 0, 0)),
            ],
            out_specs=pl.BlockSpec((K, GR, O), lambda i: (0, 0, 0)),
        ),
        compiler_params=pltpu.CompilerParams(
            dimension_semantics=("arbitrary",)),
    )(f0, w2)


def _stencil_kernel(g_ref, d8_ref, st_ref, bg_ref, bb_ref, b_ref, gavg_ref,
                    gam_ref, bet_ref, o_ref, *, W, H, PT, n_rows):
    P = W * H

    tot = jnp.sum(st_ref[...], axis=0)
    inv_n = 1.0 / float(n_rows)
    mean = tot[0:1] * inv_n
    var = tot[1:2] * inv_n - mean * mean
    y8 = d8_ref[0]
    d = jnp.tanh((y8 - mean) * lax.rsqrt(var + 1e-5) * bg_ref[...]
                 + bb_ref[...])

    iota = lax.broadcasted_iota(jnp.int32, (P, 1), 0)
    i = iota // H
    j = iota - i * H
    fi_ge1 = (i >= 1).astype(jnp.float32)
    fi_len = (i <= W - 2).astype(jnp.float32)

    y = g_ref[1, PT:PT + P, :] + b_ref[...]
    for k, lo, hi in ((0, 1, H - 1), (2, 0, H - 2)):
        dk = d[:, k:k + 1]
        xm = jnp.logical_and(j >= lo, j <= hi).astype(jnp.float32)
        dpos = jnp.maximum(dk, 0.0)
        dneg = jnp.maximum(-dk, 0.0)
        wm1 = dneg * fi_ge1 * xm
        wp1 = dpos * fi_len * xm
        w0 = 1.0 - dpos * fi_len - dneg
        w0 = jnp.where(jnp.logical_and(dk < 0, i == 0), 0.0, w0)
        w0 = jnp.where(jnp.logical_and(dk >= 1.0, i == W - 1), 0.0, w0)
        w0 = w0 * xm
        s = PT + (k - 1)
        y = y + wm1 * g_ref[k, s - H:s - H + P, :]
        y = y + w0 * g_ref[k, s:s + P, :]
        y = y + wp1 * g_ref[k, s + H:s + H + P, :]

    inv_p = 1.0 / float(P)
    cs = jnp.sum(y, axis=0, keepdims=True) * inv_p
    cq = jnp.sum(y * y, axis=0, keepdims=True) * inv_p
    gm = jnp.dot(cs, gavg_ref[...], preferred_element_type=jnp.float32)
    gq = jnp.dot(cq, gavg_ref[...], preferred_element_type=jnp.float32)
    gv = gq - gm * gm
    yn = (y - gm) * lax.rsqrt(gv + 1e-5) * gam_ref[...] + bet_ref[...]
    o_ref[0] = jnp.transpose(jnp.maximum(yn, 0.0), (1, 0))


def _stencil_gn(gpad, d8, stats, bn_g, bn_b, bias, gavg, gamma, beta,
                *, W, H, PT, NL):
    K, GR, O = gpad.shape
    B, P, _ = d8.shape
    bg = jnp.pad(bn_g, (0, NL - bn_g.shape[0])).reshape(1, NL)
    bb = jnp.pad(bn_b, (0, NL - bn_b.shape[0])).reshape(1, NL)
    b_p = bias.reshape(1, O)
    g_p = gamma.reshape(1, O)
    be_p = beta.reshape(1, O)
    kern = functools.partial(_stencil_kernel, W=W, H=H, PT=PT,
                             n_rows=B * W * H)
    return pl.pallas_call(
        kern,
        out_shape=jax.ShapeDtypeStruct((B, O, P), jnp.float32),
        grid_spec=pltpu.PrefetchScalarGridSpec(
            num_scalar_prefetch=0,
            grid=(B,),
            in_specs=[
                pl.BlockSpec((K, GR, O), lambda b: (0, 0, 0)),
                pl.BlockSpec((1, P, NL), lambda b: (b, 0, 0)),
                pl.BlockSpec((B, 8, NL), lambda b: (0, 0, 0)),
                pl.BlockSpec((1, NL), lambda b: (0, 0)),
                pl.BlockSpec((1, NL), lambda b: (0, 0)),
                pl.BlockSpec((1, O), lambda b: (0, 0)),
                pl.BlockSpec((O, O), lambda b: (0, 0)),
                pl.BlockSpec((1, O), lambda b: (0, 0)),
                pl.BlockSpec((1, O), lambda b: (0, 0)),
            ],
            out_specs=pl.BlockSpec((1, O, P), lambda b: (b, 0, 0)),
        ),
        compiler_params=pltpu.CompilerParams(
            dimension_semantics=("parallel",),
            vmem_limit_bytes=64 * 1024 * 1024),
    )(gpad, d8, stats, bg, bb, b_p, gavg, g_p, be_p)


def kernel(f, w_off, b_off, bn_gamma, bn_beta, w_x, b_x, w_y, b_y,
           gn_gamma, gn_beta):
    B, C, W, H = f.shape
    O, _, K, _ = w_x.shape
    COUT = 2 * K
    WH = W * H
    NL = _round_up(COUT, 8)
    CP = _round_up(COUT, LANE)

    fpad = jnp.pad(f, ((0, 0), (0, 0), (1, 1), (1, 1)))
    x = jnp.transpose(fpad, (0, 2, 3, 1)).reshape(B, (W + 2) * (H + 2), C)
    XR = _round_up((W + 2) * (H + 2) + 2, 8)
    x = jnp.pad(x, ((0, 0), (0, XR - x.shape[1]), (0, 0)))
    wmat = jnp.transpose(w_off, (2, 3, 1, 0)).reshape(9, C, COUT)
    wmat = jnp.pad(wmat, ((0, 0), (0, 0), (0, CP - COUT)))
    y8, stats = _conv_head(x, wmat, b_off, W=W, H=H, NL=NL)
    d8 = y8.reshape(B, W, H + 2, NL)[:, :, :H, :].reshape(B, WH, NL)

    f0 = jnp.transpose(f[0], (1, 2, 0)).reshape(WH, C)
    w2 = jnp.transpose(w_x.reshape(O, C, K), (2, 1, 0))
    PT = _round_up(H + 2, 8)
    gpad = _gtab(f0, w2, PT=PT)

    cpg = O // (O // 4)
    gids = jnp.arange(O) // cpg
    gavg = (gids[:, None] == gids[None, :]).astype(jnp.float32) / cpg
    out = _stencil_gn(gpad, d8, stats, bn_gamma, bn_beta, b_x, gavg,
                      gn_gamma, gn_beta, W=W, H=H, PT=PT, NL=NL)
    return out.reshape(B, O, W, H)


# speedup vs baseline: 1.4758x; 1.4758x over previous
import functools

import jax
import jax.numpy as jnp
from jax import lax
from jax.experimental import pallas as pl
from jax.experimental.pallas import tpu as pltpu

LANE = 128


def _round_up(x, m):
    return (x + m - 1) // m * m


def _conv_kernel(x_ref, w_ref, b_ref, y_ref, s_ref, *, W, H, NL):
    P = W * H
    PADL = _round_up(H + 1, 64)
    x = x_ref[0]
    u = jnp.dot(w_ref[...], x, preferred_element_type=jnp.float32)
    z = jnp.zeros((NL, PADL), jnp.float32)
    j = lax.broadcasted_iota(jnp.int32, (NL, P), 1) % H
    y = jnp.zeros((NL, P), jnp.float32)
    for kh in range(3):
        for kw in range(3):
            t = kh * 3 + kw
            s = (kh - 1) * H + (kw - 1)
            u_t = u[NL * t:NL * t + NL, :]
            up = jnp.concatenate([z, u_t, z], axis=1)
            sh = up[:, PADL + s:PADL + s + P]
            if kw == 0:
                sh = jnp.where(j >= 1, sh, 0.0)
            elif kw == 2:
                sh = jnp.where(j <= H - 2, sh, 0.0)
            y = y + sh
    y = y + b_ref[:, 0:1]
    s_ref[0, :, 0:1] = jnp.sum(y, axis=1, keepdims=True)
    s_ref[0, :, 1:2] = jnp.sum(y * y, axis=1, keepdims=True)
    y_ref[0] = y


def _conv_head(x, wstk, bcol, *, W, H, NL):
    B, C, P = x.shape
    kern = functools.partial(_conv_kernel, W=W, H=H, NL=NL)
    return pl.pallas_call(
        kern,
        out_shape=(jax.ShapeDtypeStruct((B, NL, P), jnp.float32),
                   jax.ShapeDtypeStruct((B, 8, 8), jnp.float32)),
        grid_spec=pltpu.PrefetchScalarGridSpec(
            num_scalar_prefetch=0,
            grid=(B,),
            in_specs=[
                pl.BlockSpec((1, C, P), lambda b: (b, 0, 0)),
                pl.BlockSpec((LANE, C), lambda b: (0, 0)),
                pl.BlockSpec((NL, LANE), lambda b: (0, 0)),
            ],
            out_specs=[pl.BlockSpec((1, NL, P), lambda b: (b, 0, 0)),
                       pl.BlockSpec((1, 8, 8), lambda b: (b, 0, 0))],
        ),
        compiler_params=pltpu.CompilerParams(
            dimension_semantics=("parallel",),
            vmem_limit_bytes=64 * 1024 * 1024),
    )(x, wstk, bcol)


def _gtab_kernel(f0_ref, w_ref, o_ref, *, PT, P):
    o_ref[...] = jnp.zeros_like(o_ref)
    f0 = f0_ref[...]
    for k in range(3):
        o_ref[k, PT:PT + P, :] = jnp.dot(f0, w_ref[k],
                                         preferred_element_type=jnp.float32)


def _gtab(f0, w2, *, PT):
    P, C = f0.shape
    K, _, O = w2.shape
    GR = P + 2 * PT
    kern = functools.partial(_gtab_kernel, PT=PT, P=P)
    return pl.pallas_call(
        kern,
        out_shape=jax.ShapeDtypeStruct((K, GR, O), jnp.float32),
        grid_spec=pltpu.PrefetchScalarGridSpec(
            num_scalar_prefetch=0,
            grid=(1,),
            in_specs=[
                pl.BlockSpec((P, C), lambda i: (0, 0)),
                pl.BlockSpec((K, C, O), lambda i: (0, 0, 0)),
            ],
            out_specs=pl.BlockSpec((K, GR, O), lambda i: (0, 0, 0)),
        ),
        compiler_params=pltpu.CompilerParams(
            dimension_semantics=("arbitrary",)),
    )(f0, w2)


def _stencil_kernel(g_ref, yc_ref, st_ref, bg_ref, bb_ref, b_ref, gavg_ref,
                    gam_ref, bet_ref, o_ref, *, W, H, PT, n_rows):
    P = W * H

    tot = jnp.sum(st_ref[...], axis=0)
    inv_n = 1.0 / float(n_rows)
    mean = tot[:, 0:1] * inv_n
    var = tot[:, 1:2] * inv_n - mean * mean
    yc = yc_ref[0]
    d8 = jnp.tanh((yc - mean) * lax.rsqrt(var + 1e-5) * bg_ref[:, 0:1]
                  + bb_ref[:, 0:1])

    iota = lax.broadcasted_iota(jnp.int32, (P, 1), 0)
    i = iota // H
    j = iota - i * H
    fi_ge1 = (i >= 1).astype(jnp.float32)
    fi_len = (i <= W - 2).astype(jnp.float32)

    y = g_ref[1, PT:PT + P, :] + b_ref[...]
    for k, lo, hi in ((0, 1, H - 1), (2, 0, H - 2)):
        dk = jnp.transpose(d8[k:k + 1, :], (1, 0))
        xm = jnp.logical_and(j >= lo, j <= hi).astype(jnp.float32)
        dpos = jnp.maximum(dk, 0.0)
        dneg = jnp.maximum(-dk, 0.0)
        wm1 = dneg * fi_ge1 * xm
        wp1 = dpos * fi_len * xm
        w0 = 1.0 - dpos * fi_len - dneg
        w0 = jnp.where(jnp.logical_and(dk < 0, i == 0), 0.0, w0)
        w0 = jnp.where(jnp.logical_and(dk >= 1.0, i == W - 1), 0.0, w0)
        w0 = w0 * xm
        s = PT + (k - 1)
        y = y + wm1 * g_ref[k, s - H:s - H + P, :]
        y = y + w0 * g_ref[k, s:s + P, :]
        y = y + wp1 * g_ref[k, s + H:s + H + P, :]

    inv_p = 1.0 / float(P)
    cs = jnp.sum(y, axis=0, keepdims=True) * inv_p
    cq = jnp.sum(y * y, axis=0, keepdims=True) * inv_p
    gm = jnp.dot(cs, gavg_ref[...], preferred_element_type=jnp.float32)
    gq = jnp.dot(cq, gavg_ref[...], preferred_element_type=jnp.float32)
    gv = gq - gm * gm
    yn = (y - gm) * lax.rsqrt(gv + 1e-5) * gam_ref[...] + bet_ref[...]
    o_ref[0] = jnp.transpose(jnp.maximum(yn, 0.0), (1, 0))


def _stencil_gn(gpad, yc, stats, bgc, bbc, bias, gavg, gamma, beta,
                *, W, H, PT):
    K, GR, O = gpad.shape
    B, NL, P = yc.shape
    b_p = bias.reshape(1, O)
    g_p = gamma.reshape(1, O)
    be_p = beta.reshape(1, O)
    kern = functools.partial(_stencil_kernel, W=W, H=H, PT=PT,
                             n_rows=B * W * H)
    return pl.pallas_call(
        kern,
        out_shape=jax.ShapeDtypeStruct((B, O, P), jnp.float32),
        grid_spec=pltpu.PrefetchScalarGridSpec(
            num_scalar_prefetch=0,
            grid=(B,),
            in_specs=[
                pl.BlockSpec((K, GR, O), lambda b: (0, 0, 0)),
                pl.BlockSpec((1, NL, P), lambda b: (b, 0, 0)),
                pl.BlockSpec((B, 8, 8), lambda b: (0, 0, 0)),
                pl.BlockSpec((NL, LANE), lambda b: (0, 0)),
                pl.BlockSpec((NL, LANE), lambda b: (0, 0)),
                pl.BlockSpec((1, O), lambda b: (0, 0)),
                pl.BlockSpec((O, O), lambda b: (0, 0)),
                pl.BlockSpec((1, O), lambda b: (0, 0)),
                pl.BlockSpec((1, O), lambda b: (0, 0)),
            ],
            out_specs=pl.BlockSpec((1, O, P), lambda b: (b, 0, 0)),
        ),
        compiler_params=pltpu.CompilerParams(
            dimension_semantics=("parallel",),
            vmem_limit_bytes=64 * 1024 * 1024),
    )(gpad, yc, stats, bgc, bbc, b_p, gavg, g_p, be_p)


def _col128(v, NL):
    c = jnp.pad(v, (0, NL - v.shape[0])).reshape(NL, 1)
    return jnp.pad(c, ((0, 0), (0, LANE - 1)))


def kernel(f, w_off, b_off, bn_gamma, bn_beta, w_x, b_x, w_y, b_y,
           gn_gamma, gn_beta):
    B, C, W, H = f.shape
    O, _, K, _ = w_x.shape
    COUT = 2 * K
    WH = W * H
    NL = _round_up(COUT, 8)

    x = f.reshape(B, C, WH)
    wstk = jnp.transpose(w_off, (2, 3, 0, 1)).reshape(9, COUT, C)
    wstk = jnp.pad(wstk, ((0, 0), (0, NL - COUT), (0, 0))).reshape(9 * NL, C)
    wstk = jnp.pad(wstk, ((0, LANE - 9 * NL), (0, 0)))
    yc, stats = _conv_head(x, wstk, _col128(b_off, NL), W=W, H=H, NL=NL)

    f0 = jnp.transpose(f[0], (1, 2, 0)).reshape(WH, C)
    w2 = jnp.transpose(w_x.reshape(O, C, K), (2, 1, 0))
    PT = _round_up(H + 2, 8)
    gpad = _gtab(f0, w2, PT=PT)

    cpg = O // (O // 4)
    gids = jnp.arange(O) // cpg
    gavg = (gids[:, None] == gids[None, :]).astype(jnp.float32) / cpg
    out = _stencil_gn(gpad, yc, stats, _col128(bn_gamma, NL),
                      _col128(bn_beta, NL), b_x, gavg, gn_gamma, gn_beta,
                      W=W, H=H, PT=PT)
    return out.reshape(B, O, W, H)


# speedup vs baseline: 1.5499x; 1.0502x over previous
import functools

import jax
import jax.numpy as jnp
from jax import lax
from jax.experimental import pallas as pl
from jax.experimental.pallas import tpu as pltpu

LANE = 128


def _round_up(x, m):
    return (x + m - 1) // m * m


def _conv_kernel(x_ref, w_ref, b_ref, y_ref, s_ref, *, W, H, NL):
    P = W * H
    PADL = _round_up(H + 1, 64)
    x = x_ref[0]
    u = jnp.dot(w_ref[...], x, preferred_element_type=jnp.float32)
    z = jnp.zeros((NL, PADL), jnp.float32)
    j = lax.broadcasted_iota(jnp.int32, (NL, P), 1) % H
    y = jnp.zeros((NL, P), jnp.float32)
    for kh in range(3):
        for kw in range(3):
            t = kh * 3 + kw
            s = (kh - 1) * H + (kw - 1)
            u_t = u[NL * t:NL * t + NL, :]
            up = jnp.concatenate([z, u_t, z], axis=1)
            sh = up[:, PADL + s:PADL + s + P]
            if kw == 0:
                sh = jnp.where(j >= 1, sh, 0.0)
            elif kw == 2:
                sh = jnp.where(j <= H - 2, sh, 0.0)
            y = y + sh
    y = y + b_ref[:, 0:1]
    s_ref[0, :, 0:1] = jnp.sum(y, axis=1, keepdims=True)
    s_ref[0, :, 1:2] = jnp.sum(y * y, axis=1, keepdims=True)
    y_ref[0] = y


def _conv_head(x, wstk, bcol, *, W, H, NL):
    B, C, P = x.shape
    kern = functools.partial(_conv_kernel, W=W, H=H, NL=NL)
    return pl.pallas_call(
        kern,
        out_shape=(jax.ShapeDtypeStruct((B, NL, P), jnp.float32),
                   jax.ShapeDtypeStruct((B, 8, 8), jnp.float32)),
        grid_spec=pltpu.PrefetchScalarGridSpec(
            num_scalar_prefetch=0,
            grid=(B,),
            in_specs=[
                pl.BlockSpec((1, C, P), lambda b: (b, 0, 0)),
                pl.BlockSpec((LANE, C), lambda b: (0, 0)),
                pl.BlockSpec((NL, LANE), lambda b: (0, 0)),
            ],
            out_specs=[pl.BlockSpec((1, NL, P), lambda b: (b, 0, 0)),
                       pl.BlockSpec((1, 8, 8), lambda b: (b, 0, 0))],
        ),
        compiler_params=pltpu.CompilerParams(
            dimension_semantics=("parallel",),
            vmem_limit_bytes=64 * 1024 * 1024),
    )(x, wstk, bcol)


def _gtab_kernel(f0_ref, w_ref, o_ref, *, PT, P):
    o_ref[...] = jnp.zeros_like(o_ref)
    f0 = f0_ref[...]
    for k in range(3):
        o_ref[k, PT:PT + P, :] = jnp.dot(f0, w_ref[k],
                                         preferred_element_type=jnp.float32)


def _gtab(f0, w2, *, PT):
    P, C = f0.shape
    K, _, O = w2.shape
    GR = P + 2 * PT
    kern = functools.partial(_gtab_kernel, PT=PT, P=P)
    return pl.pallas_call(
        kern,
        out_shape=jax.ShapeDtypeStruct((K, GR, O), jnp.float32),
        grid_spec=pltpu.PrefetchScalarGridSpec(
            num_scalar_prefetch=0,
            grid=(1,),
            in_specs=[
                pl.BlockSpec((P, C), lambda i: (0, 0)),
                pl.BlockSpec((K, C, O), lambda i: (0, 0, 0)),
            ],
            out_specs=pl.BlockSpec((K, GR, O), lambda i: (0, 0, 0)),
        ),
        compiler_params=pltpu.CompilerParams(
            dimension_semantics=("arbitrary",)),
    )(f0, w2)


def _stencil_kernel(g_ref, yc_ref, st_ref, bg_ref, bb_ref, b_ref, gavg_ref,
                    gam_ref, bet_ref, o_ref, *, W, H, PT, n_rows, BB):
    P = W * H

    tot = jnp.sum(st_ref[...], axis=0)
    inv_n = 1.0 / float(n_rows)
    mean = tot[:, 0:1] * inv_n
    var = tot[:, 1:2] * inv_n - mean * mean
    bn_s = lax.rsqrt(var + 1e-5) * bg_ref[:, 0:1]

    iota = lax.broadcasted_iota(jnp.int32, (P, 1), 0)
    i = iota // H
    j = iota - i * H
    fi_ge1 = (i >= 1).astype(jnp.float32)
    fi_len = (i <= W - 2).astype(jnp.float32)

    gc = g_ref[1, PT:PT + P, :] + b_ref[...]
    gs = [[g_ref[k, PT + (k - 1) + dr * H:PT + (k - 1) + dr * H + P, :]
           for dr in (-1, 0, 1)] for k in (0, 2)]

    for bb in range(BB):
        yc = yc_ref[bb]
        d8 = jnp.tanh((yc - mean) * bn_s + bb_ref[:, 0:1])
        y = gc
        for kk, (k, lo, hi) in enumerate(((0, 1, H - 1), (2, 0, H - 2))):
            dk = jnp.transpose(d8[k:k + 1, :], (1, 0))
            xm = jnp.logical_and(j >= lo, j <= hi).astype(jnp.float32)
            dpos = jnp.maximum(dk, 0.0)
            dneg = jnp.maximum(-dk, 0.0)
            wm1 = dneg * fi_ge1 * xm
            wp1 = dpos * fi_len * xm
            w0 = 1.0 - dpos * fi_len - dneg
            w0 = jnp.where(jnp.logical_and(dk < 0, i == 0), 0.0, w0)
            w0 = jnp.where(jnp.logical_and(dk >= 1.0, i == W - 1), 0.0, w0)
            w0 = w0 * xm
            y = y + wm1 * gs[kk][0] + w0 * gs[kk][1] + wp1 * gs[kk][2]

        inv_p = 1.0 / float(P)
        cs = jnp.sum(y, axis=0, keepdims=True) * inv_p
        cq = jnp.sum(y * y, axis=0, keepdims=True) * inv_p
        gm = jnp.dot(cs, gavg_ref[...], preferred_element_type=jnp.float32)
        gq = jnp.dot(cq, gavg_ref[...], preferred_element_type=jnp.float32)
        gv = gq - gm * gm
        yn = (y - gm) * lax.rsqrt(gv + 1e-5) * gam_ref[...] + bet_ref[...]
        o_ref[bb] = jnp.transpose(jnp.maximum(yn, 0.0), (1, 0))


def _stencil_gn(gpad, yc, stats, bgc, bbc, bias, gavg, gamma, beta,
                *, W, H, PT):
    K, GR, O = gpad.shape
    B, NL, P = yc.shape
    b_p = bias.reshape(1, O)
    g_p = gamma.reshape(1, O)
    be_p = beta.reshape(1, O)
    BB = 2 if B % 2 == 0 else 1
    kern = functools.partial(_stencil_kernel, W=W, H=H, PT=PT,
                             n_rows=B * W * H, BB=BB)
    return pl.pallas_call(
        kern,
        out_shape=jax.ShapeDtypeStruct((B, O, P), jnp.float32),
        grid_spec=pltpu.PrefetchScalarGridSpec(
            num_scalar_prefetch=0,
            grid=(B // BB,),
            in_specs=[
                pl.BlockSpec((K, GR, O), lambda b: (0, 0, 0)),
                pl.BlockSpec((BB, NL, P), lambda b: (b, 0, 0)),
                pl.BlockSpec((B, 8, 8), lambda b: (0, 0, 0)),
                pl.BlockSpec((NL, LANE), lambda b: (0, 0)),
                pl.BlockSpec((NL, LANE), lambda b: (0, 0)),
                pl.BlockSpec((1, O), lambda b: (0, 0)),
                pl.BlockSpec((O, O), lambda b: (0, 0)),
                pl.BlockSpec((1, O), lambda b: (0, 0)),
                pl.BlockSpec((1, O), lambda b: (0, 0)),
            ],
            out_specs=pl.BlockSpec((BB, O, P), lambda b: (b, 0, 0)),
        ),
        compiler_params=pltpu.CompilerParams(
            dimension_semantics=("parallel",),
            vmem_limit_bytes=64 * 1024 * 1024),
    )(gpad, yc, stats, bgc, bbc, b_p, gavg, g_p, be_p)


def _col128(v, NL):
    c = jnp.pad(v, (0, NL - v.shape[0])).reshape(NL, 1)
    return jnp.pad(c, ((0, 0), (0, LANE - 1)))


def kernel(f, w_off, b_off, bn_gamma, bn_beta, w_x, b_x, w_y, b_y,
           gn_gamma, gn_beta):
    B, C, W, H = f.shape
    O, _, K, _ = w_x.shape
    COUT = 2 * K
    WH = W * H
    NL = _round_up(COUT, 8)

    x = f.reshape(B, C, WH)
    wstk = jnp.transpose(w_off, (2, 3, 0, 1)).reshape(9, COUT, C)
    wstk = jnp.pad(wstk, ((0, 0), (0, NL - COUT), (0, 0))).reshape(9 * NL, C)
    wstk = jnp.pad(wstk, ((0, LANE - 9 * NL), (0, 0)))
    yc, stats = _conv_head(x, wstk, _col128(b_off, NL), W=W, H=H, NL=NL)

    f0 = jnp.transpose(f[0], (1, 2, 0)).reshape(WH, C)
    w2 = jnp.transpose(w_x.reshape(O, C, K), (2, 1, 0))
    PT = _round_up(H + 2, 8)
    gpad = _gtab(f0, w2, PT=PT)

    cpg = O // (O // 4)
    gids = jnp.arange(O) // cpg
    gavg = (gids[:, None] == gids[None, :]).astype(jnp.float32) / cpg
    out = _stencil_gn(gpad, yc, stats, _col128(bn_gamma, NL),
                      _col128(bn_beta, NL), b_x, gavg, gn_gamma, gn_beta,
                      W=W, H=H, PT=PT)
    return out.reshape(B, O, W, H)


# speedup vs baseline: 1.5855x; 1.0229x over previous
import functools

import jax
import jax.numpy as jnp
from jax import lax
from jax.experimental import pallas as pl
from jax.experimental.pallas import tpu as pltpu

LANE = 128


def _round_up(x, m):
    return (x + m - 1) // m * m


def _conv_kernel(x_ref, w_ref, b_ref, y_ref, s_ref, *, W, H, NL):
    P = W * H
    PADL = _round_up(H + 1, 64)
    x = x_ref[0]
    u = jnp.dot(w_ref[...], x, preferred_element_type=jnp.float32)
    z = jnp.zeros((NL, PADL), jnp.float32)
    j = lax.broadcasted_iota(jnp.int32, (NL, P), 1) % H
    y = jnp.zeros((NL, P), jnp.float32)
    for kh in range(3):
        for kw in range(3):
            t = kh * 3 + kw
            s = (kh - 1) * H + (kw - 1)
            u_t = u[NL * t:NL * t + NL, :]
            up = jnp.concatenate([z, u_t, z], axis=1)
            sh = up[:, PADL + s:PADL + s + P]
            if kw == 0:
                sh = jnp.where(j >= 1, sh, 0.0)
            elif kw == 2:
                sh = jnp.where(j <= H - 2, sh, 0.0)
            y = y + sh
    y = y + b_ref[:, 0:1]
    s_ref[0, :, 0:1] = jnp.sum(y, axis=1, keepdims=True)
    s_ref[0, :, 1:2] = jnp.sum(y * y, axis=1, keepdims=True)
    y_ref[0] = y


def _conv_head(x, wstk, bcol, *, W, H, NL):
    B, C, P = x.shape
    kern = functools.partial(_conv_kernel, W=W, H=H, NL=NL)
    return pl.pallas_call(
        kern,
        out_shape=(jax.ShapeDtypeStruct((B, NL, P), jnp.float32),
                   jax.ShapeDtypeStruct((B, 8, 8), jnp.float32)),
        grid_spec=pltpu.PrefetchScalarGridSpec(
            num_scalar_prefetch=0,
            grid=(B,),
            in_specs=[
                pl.BlockSpec((1, C, P), lambda b: (b, 0, 0)),
                pl.BlockSpec((LANE, C), lambda b: (0, 0)),
                pl.BlockSpec((NL, LANE), lambda b: (0, 0)),
            ],
            out_specs=[pl.BlockSpec((1, NL, P), lambda b: (b, 0, 0)),
                       pl.BlockSpec((1, 8, 8), lambda b: (b, 0, 0))],
        ),
        compiler_params=pltpu.CompilerParams(
            dimension_semantics=("parallel",),
            vmem_limit_bytes=64 * 1024 * 1024),
    )(x, wstk, bcol)


def _gtab_kernel(f0_ref, w_ref, o_ref, *, PT, P):
    o_ref[...] = jnp.zeros_like(o_ref)
    f0 = f0_ref[...]
    for k in range(3):
        o_ref[k, PT:PT + P, :] = jnp.dot(f0, w_ref[k],
                                         preferred_element_type=jnp.float32)


def _gtab(f0, w2, *, PT):
    P, C = f0.shape
    K, _, O = w2.shape
    GR = P + 2 * PT
    kern = functools.partial(_gtab_kernel, PT=PT, P=P)
    return pl.pallas_call(
        kern,
        out_shape=jax.ShapeDtypeStruct((K, GR, O), jnp.float32),
        grid_spec=pltpu.PrefetchScalarGridSpec(
            num_scalar_prefetch=0,
            grid=(1,),
            in_specs=[
                pl.BlockSpec((P, C), lambda i: (0, 0)),
                pl.BlockSpec((K, C, O), lambda i: (0, 0, 0)),
            ],
            out_specs=pl.BlockSpec((K, GR, O), lambda i: (0, 0, 0)),
        ),
        compiler_params=pltpu.CompilerParams(
            dimension_semantics=("arbitrary",)),
    )(f0, w2)


def _stencil_kernel(g_ref, yc_ref, st_ref, bg_ref, bb_ref, b_ref, gavg_ref,
                    gam_ref, bet_ref, o_ref, *, W, H, PT, n_rows, BB):
    P = W * H

    tot = jnp.sum(st_ref[...], axis=0)
    inv_n = 1.0 / float(n_rows)
    mean = tot[:, 0:1] * inv_n
    var = tot[:, 1:2] * inv_n - mean * mean
    bn_s = lax.rsqrt(var + 1e-5) * bg_ref[:, 0:1]

    iota = lax.broadcasted_iota(jnp.int32, (P, 1), 0)
    i = iota // H
    j = iota - i * H
    fi_ge1 = (i >= 1).astype(jnp.float32)
    fi_len = (i <= W - 2).astype(jnp.float32)

    gc = g_ref[1, PT:PT + P, :] + b_ref[...]
    gs = [[g_ref[k, PT + (k - 1) + dr * H:PT + (k - 1) + dr * H + P, :]
           for dr in (-1, 0, 1)] for k in (0, 2)]

    for bb in range(BB):
        yc = yc_ref[bb]
        d8 = jnp.tanh((yc - mean) * bn_s + bb_ref[:, 0:1])
        y = gc
        for kk, (k, lo, hi) in enumerate(((0, 1, H - 1), (2, 0, H - 2))):
            dk = jnp.transpose(d8[k:k + 1, :], (1, 0))
            xm = jnp.logical_and(j >= lo, j <= hi).astype(jnp.float32)
            dpos = jnp.maximum(dk, 0.0)
            dneg = jnp.maximum(-dk, 0.0)
            wm1 = dneg * fi_ge1 * xm
            wp1 = dpos * fi_len * xm
            w0 = 1.0 - dpos * fi_len - dneg
            w0 = jnp.where(jnp.logical_and(dk < 0, i == 0), 0.0, w0)
            w0 = jnp.where(jnp.logical_and(dk >= 1.0, i == W - 1), 0.0, w0)
            w0 = w0 * xm
            y = y + wm1 * gs[kk][0] + w0 * gs[kk][1] + wp1 * gs[kk][2]

        inv_p = 1.0 / float(P)
        cs = jnp.sum(y, axis=0, keepdims=True) * inv_p
        cq = jnp.sum(y * y, axis=0, keepdims=True) * inv_p
        gm = jnp.dot(cs, gavg_ref[...], preferred_element_type=jnp.float32)
        gq = jnp.dot(cq, gavg_ref[...], preferred_element_type=jnp.float32)
        gv = gq - gm * gm
        yn = (y - gm) * lax.rsqrt(gv + 1e-5) * gam_ref[...] + bet_ref[...]
        o_ref[bb] = jnp.transpose(jnp.maximum(yn, 0.0), (1, 0))


def _stencil_gn(gpad, yc, stats, bgc, bbc, bias, gavg, gamma, beta,
                *, W, H, PT):
    K, GR, O = gpad.shape
    B, NL, P = yc.shape
    b_p = bias.reshape(1, O)
    g_p = gamma.reshape(1, O)
    be_p = beta.reshape(1, O)
    BB = 4 if B % 4 == 0 else (2 if B % 2 == 0 else 1)
    kern = functools.partial(_stencil_kernel, W=W, H=H, PT=PT,
                             n_rows=B * W * H, BB=BB)
    return pl.pallas_call(
        kern,
        out_shape=jax.ShapeDtypeStruct((B, O, P), jnp.float32),
        grid_spec=pltpu.PrefetchScalarGridSpec(
            num_scalar_prefetch=0,
            grid=(B // BB,),
            in_specs=[
                pl.BlockSpec((K, GR, O), lambda b: (0, 0, 0)),
                pl.BlockSpec((BB, NL, P), lambda b: (b, 0, 0)),
                pl.BlockSpec((B, 8, 8), lambda b: (0, 0, 0)),
                pl.BlockSpec((NL, LANE), lambda b: (0, 0)),
                pl.BlockSpec((NL, LANE), lambda b: (0, 0)),
                pl.BlockSpec((1, O), lambda b: (0, 0)),
                pl.BlockSpec((O, O), lambda b: (0, 0)),
                pl.BlockSpec((1, O), lambda b: (0, 0)),
                pl.BlockSpec((1, O), lambda b: (0, 0)),
            ],
            out_specs=pl.BlockSpec((BB, O, P), lambda b: (b, 0, 0)),
        ),
        compiler_params=pltpu.CompilerParams(
            dimension_semantics=("parallel",),
            vmem_limit_bytes=64 * 1024 * 1024),
    )(gpad, yc, stats, bgc, bbc, b_p, gavg, g_p, be_p)


def _col128(v, NL):
    c = jnp.pad(v, (0, NL - v.shape[0])).reshape(NL, 1)
    return jnp.pad(c, ((0, 0), (0, LANE - 1)))


def kernel(f, w_off, b_off, bn_gamma, bn_beta, w_x, b_x, w_y, b_y,
           gn_gamma, gn_beta):
    B, C, W, H = f.shape
    O, _, K, _ = w_x.shape
    COUT = 2 * K
    WH = W * H
    NL = _round_up(COUT, 8)

    x = f.reshape(B, C, WH)
    wstk = jnp.transpose(w_off, (2, 3, 0, 1)).reshape(9, COUT, C)
    wstk = jnp.pad(wstk, ((0, 0), (0, NL - COUT), (0, 0))).reshape(9 * NL, C)
    wstk = jnp.pad(wstk, ((0, LANE - 9 * NL), (0, 0)))
    yc, stats = _conv_head(x, wstk, _col128(b_off, NL), W=W, H=H, NL=NL)

    f0 = jnp.transpose(f[0], (1, 2, 0)).reshape(WH, C)
    w2 = jnp.transpose(w_x.reshape(O, C, K), (2, 1, 0))
    PT = _round_up(H + 2, 8)
    gpad = _gtab(f0, w2, PT=PT)

    cpg = O // (O // 4)
    gids = jnp.arange(O) // cpg
    gavg = (gids[:, None] == gids[None, :]).astype(jnp.float32) / cpg
    out = _stencil_gn(gpad, yc, stats, _col128(bn_gamma, NL),
                      _col128(bn_beta, NL), b_x, gavg, gn_gamma, gn_beta,
                      W=W, H=H, PT=PT)
    return out.reshape(B, O, W, H)
